# baseline (speedup 1.0000x reference)
import functools

import jax
import jax.numpy as jnp
from jax import lax
from jax.experimental import pallas as pl
from jax.experimental.pallas import tpu as pltpu
from jax.experimental.pallas import tpu_sc as plsc

_NBUF = 4


def _make_gather(B, L, D):
    info = plsc.get_sparse_core_info()
    NC, NS = info.num_cores, info.num_subcores
    NW = NC * NS
    s_per_w = B // NW
    n_groups = s_per_w // _NBUF
    assert B % NW == 0 and s_per_w % _NBUF == 0
    mesh = plsc.VectorSubcoreMesh(core_axis_name="c", subcore_axis_name="s")

    @functools.partial(
        pl.kernel,
        mesh=mesh,
        out_type=jax.ShapeDtypeStruct((B, L, D), jnp.float32),
        scratch_types=(
            [pltpu.VMEM((s_per_w, L), jnp.int32)]
            + [pltpu.VMEM((L, D), jnp.float32) for _ in range(_NBUF)]
            + [pltpu.SemaphoreType.DMA for _ in range(2 * _NBUF)]
        ),
        compiler_params=pltpu.CompilerParams(use_tc_tiling_on_sc=False),
    )
    def gather_kernel(idx_hbm, table_hbm, out_hbm, idx_all, *bufs):
        rows = bufs[:_NBUF]
        gsem = bufs[_NBUF : 2 * _NBUF]
        osem = bufs[2 * _NBUF :]
        wid = lax.axis_index("s") * NC + lax.axis_index("c")
        base = wid * s_per_w
        pltpu.sync_copy(idx_hbm.at[pl.ds(base, s_per_w), :], idx_all)

        def start_gather(j, b):
            pltpu.async_copy(table_hbm.at[idx_all.at[j]], rows[b], gsem[b])

        def wait_gather(j, b):
            pltpu.make_async_copy(
                table_hbm.at[idx_all.at[j]], rows[b], gsem[b]
            ).wait()

        def start_out(j, b):
            pltpu.async_copy(rows[b], out_hbm.at[base + j], osem[b])

        def wait_out(j, b):
            pltpu.make_async_copy(rows[b], out_hbm.at[base + j], osem[b]).wait()

        start_gather(0, 0)
        start_gather(1, 1)

        def group(g, carry):
            for b in range(_NBUF):
                j = g * _NBUF + b
                b2 = (b + 2) % _NBUF
                wait_gather(j, b)
                start_out(j, b)

                @pl.when(j + 2 < s_per_w)
                def _():
                    @pl.when(j + 2 >= _NBUF)
                    def _():
                        wait_out(j + 2 - _NBUF, b2)

                    start_gather(j + 2, b2)

            return carry

        lax.fori_loop(0, n_groups, group, 0)
        for b in range(_NBUF):
            wait_out(s_per_w - _NBUF + b, b)

    return gather_kernel


def kernel(x, table):
    B, L = x.shape
    V, D = table.shape
    return _make_gather(B, L, D)(x.astype(jnp.int32), table)
